# Initial kernel scaffold; baseline (speedup 1.0000x reference)
import jax
import jax.numpy as jnp
from jax.experimental import pallas as pl


def kernel(x, table, W, b):
    raise NotImplementedError("write your pallas kernel here")


# speedup vs baseline: 7.3302x; 7.3302x over previous
import functools

import jax
import jax.numpy as jnp
from jax import lax
from jax.experimental import pallas as pl
from jax.experimental.pallas import tpu as pltpu
from jax.experimental.pallas import tpu_sc as plsc

VOCAB = 1000000
EMBED_DIM = 16
BATCH = 16384
HIST = 200

_PACK = 128 // EMBED_DIM
_ROWS = VOCAB // _PACK
_BLK = 1000
_GRID = _ROWS // _BLK

_NC = 2
_NS = 16
_NW = _NC * _NS
_RB = 16
_NB = BATCH // _RB
_BPW = _NB // _NW
_IDX_ROWS = HIST * _RB // 128


def _score_body(m_ref, s_ref, b_ref, o_ref):
    o_ref[...] = (
        jnp.dot(m_ref[...], s_ref[...], preferred_element_type=jnp.float32)
        + b_ref[0, 0]
    )


_score = pl.pallas_call(
    _score_body,
    grid=(_GRID,),
    in_specs=[
        pl.BlockSpec((_BLK, 128), lambda i: (i, 0)),
        pl.BlockSpec((128, _PACK), lambda i: (0, 0)),
        pl.BlockSpec(memory_space=pltpu.SMEM),
    ],
    out_specs=pl.BlockSpec((_BLK, _PACK), lambda i: (i, 0)),
    out_shape=jax.ShapeDtypeStruct((_ROWS, _PACK), jnp.float32),
)


def _pool_body(xr_hbm, s_hbm, out_hbm, idx_v, g_v, out_v, sem):
    wid = lax.axis_index("s") * _NC + lax.axis_index("c")
    base = wid * _BPW

    def body(blk, carry):
        gb = base + blk
        pltpu.sync_copy(xr_hbm.at[gb], idx_v)
        copies = []
        for i in range(_IDX_ROWS):
            copies.append(pltpu.async_copy(
                s_hbm.at[idx_v.at[pl.ds(i * 128, 128)]],
                g_v.at[pl.ds(i * 128, 128)],
                sem,
            ))
        for cpy in copies:
            cpy.wait()
        acc = jnp.zeros((16,), jnp.float32)
        for i in range(_IDX_ROWS):
            for c in range(_PACK):
                acc = acc + g_v[pl.ds(i * 128 + c * 16, 16)]
        m = acc * (1.0 / HIST)
        y = 1.0 / (1.0 + jnp.exp(-m))
        y = (y * 100.0 + 0.5).astype(jnp.int32).astype(jnp.float32) / 100.0
        out_v[pl.ds(blk * _RB, _RB)] = y
        return carry

    lax.fori_loop(0, _BPW, body, 0)
    pltpu.sync_copy(out_v, out_hbm.at[pl.ds(base * _RB, _BPW * _RB)])


@functools.cache
def _pool():
    return pl.kernel(
        _pool_body,
        out_type=jax.ShapeDtypeStruct((BATCH,), jnp.float32),
        mesh=plsc.VectorSubcoreMesh(
            core_axis_name="c", subcore_axis_name="s",
            num_cores=_NC, num_subcores=_NS,
        ),
        scratch_types=[
            pltpu.VMEM((_IDX_ROWS * 128,), jnp.int32),
            pltpu.VMEM((_IDX_ROWS * 128,), jnp.float32),
            pltpu.VMEM((_BPW * _RB,), jnp.float32),
            pltpu.SemaphoreType.DMA,
        ],
    )


def kernel(x, table, W, b):
    xr = (
        x.astype(jnp.int32)
        .reshape(_NB, _RB, HIST)
        .transpose(0, 2, 1)
        .reshape(_NB, _IDX_ROWS * 128)
    )
    sel = jnp.repeat(jnp.eye(_PACK, dtype=jnp.float32), EMBED_DIM, axis=0)
    sel = sel * jnp.tile(W.reshape(EMBED_DIM), _PACK)[:, None]
    scores = _score(table.reshape(_ROWS, 128), sel, b.reshape(1, 1))
    y = _pool()(xr, scores.reshape(VOCAB))
    return y.reshape(BATCH, 1)


# speedup vs baseline: 7.5061x; 1.0240x over previous
import functools

import jax
import jax.numpy as jnp
from jax import lax
from jax.experimental import pallas as pl
from jax.experimental.pallas import tpu as pltpu
from jax.experimental.pallas import tpu_sc as plsc

VOCAB = 1000000
EMBED_DIM = 16
BATCH = 16384
HIST = 200

_PACK = 128 // EMBED_DIM
_ROWS = VOCAB // _PACK
_BLK = 1000
_GRID = _ROWS // _BLK

_NC = 2
_NS = 16
_NW = _NC * _NS
_RB = 16
_NB = BATCH // _RB
_BPW = _NB // _NW
_IDX_ROWS = HIST * _RB // 128


def _score_body(m_ref, s_ref, b_ref, o_ref):
    o_ref[...] = (
        jnp.dot(m_ref[...], s_ref[...], preferred_element_type=jnp.float32)
        + b_ref[0, 0]
    )


_score = pl.pallas_call(
    _score_body,
    grid=(_GRID,),
    in_specs=[
        pl.BlockSpec((_BLK, 128), lambda i: (i, 0)),
        pl.BlockSpec((128, _PACK), lambda i: (0, 0)),
        pl.BlockSpec(memory_space=pltpu.SMEM),
    ],
    out_specs=pl.BlockSpec((_BLK, _PACK), lambda i: (i, 0)),
    out_shape=jax.ShapeDtypeStruct((_ROWS, _PACK), jnp.float32),
)


_CHUNK = _IDX_ROWS * 128
_NBUF = 2


def _pool_body(xr_hbm, s_hbm, out_hbm, idx_v, g0, g1, out_v, sem0, sem1):
    g = [g0, g1]
    sem = [sem0, sem1]
    wid = lax.axis_index("s") * _NC + lax.axis_index("c")
    base = wid * _BPW

    pltpu.sync_copy(xr_hbm.at[pl.ds(base * _CHUNK, _BPW * _CHUNK)], idx_v)

    def fire(blk, k):
        off = blk * _CHUNK
        for i in range(_IDX_ROWS):
            pltpu.async_copy(
                s_hbm.at[idx_v.at[pl.ds(off + i * 128, 128)]],
                g[k].at[pl.ds(i * 128, 128)],
                sem[k],
            )

    def wait_g(k):
        pltpu.make_async_copy(s_hbm.at[pl.ds(0, _CHUNK)], g[k], sem[k]).wait()

    for k in range(_NBUF):
        fire(k, k)

    def body(it, carry):
        blk0 = it * _NBUF
        for k in range(_NBUF):
            blk = blk0 + k
            wait_g(k)
            acc = jnp.zeros((16,), jnp.float32)
            for i in range(_IDX_ROWS):
                for c in range(_PACK):
                    acc = acc + g[k][pl.ds(i * 128 + c * 16, 16)]
            m = acc * (1.0 / HIST)
            y = 1.0 / (1.0 + jnp.exp(-m))
            y = (y * 100.0 + 0.5).astype(jnp.int32).astype(jnp.float32) / 100.0
            out_v[pl.ds(blk * _RB, _RB)] = y
            nxt = blk + _NBUF

            @pl.when(nxt < _BPW)
            def _():
                fire(nxt, k)

        return carry

    lax.fori_loop(0, _BPW // _NBUF, body, 0)
    pltpu.sync_copy(out_v, out_hbm.at[pl.ds(base * _RB, _BPW * _RB)])


@functools.cache
def _pool():
    return pl.kernel(
        _pool_body,
        out_type=jax.ShapeDtypeStruct((BATCH,), jnp.float32),
        mesh=plsc.VectorSubcoreMesh(
            core_axis_name="c", subcore_axis_name="s",
            num_cores=_NC, num_subcores=_NS,
        ),
        scratch_types=[
            pltpu.VMEM((_BPW * _CHUNK,), jnp.int32),
            pltpu.VMEM((_CHUNK,), jnp.float32),
            pltpu.VMEM((_CHUNK,), jnp.float32),
            pltpu.VMEM((_BPW * _RB,), jnp.float32),
            pltpu.SemaphoreType.DMA,
            pltpu.SemaphoreType.DMA,
        ],
    )


def kernel(x, table, W, b):
    xr = (
        x.astype(jnp.int32)
        .reshape(_NB, _RB, HIST)
        .transpose(0, 2, 1)
        .reshape(_NB * _CHUNK)
    )
    sel = jnp.repeat(jnp.eye(_PACK, dtype=jnp.float32), EMBED_DIM, axis=0)
    sel = sel * jnp.tile(W.reshape(EMBED_DIM), _PACK)[:, None]
    scores = _score(table.reshape(_ROWS, 128), sel, b.reshape(1, 1))
    y = _pool()(xr, scores.reshape(VOCAB))
    return y.reshape(BATCH, 1)


# speedup vs baseline: 7.5118x; 1.0008x over previous
import functools

import jax
import jax.numpy as jnp
from jax import lax
from jax.experimental import pallas as pl
from jax.experimental.pallas import tpu as pltpu
from jax.experimental.pallas import tpu_sc as plsc

VOCAB = 1000000
EMBED_DIM = 16
BATCH = 16384
HIST = 200

_PACK = 128 // EMBED_DIM
_ROWS = VOCAB // _PACK
_BLK = 1000
_GRID = _ROWS // _BLK

_NC = 2
_NS = 16
_NW = _NC * _NS
_RB = 16
_NB = BATCH // _RB
_BPW = _NB // _NW
_IDX_ROWS = HIST * _RB // 128


def _score_body(m_ref, s_ref, b_ref, o_ref):
    o_ref[...] = (
        jnp.dot(m_ref[...], s_ref[...], preferred_element_type=jnp.float32)
        + b_ref[0, 0]
    )


_score = pl.pallas_call(
    _score_body,
    grid=(_GRID,),
    in_specs=[
        pl.BlockSpec((_BLK, 128), lambda i: (i, 0)),
        pl.BlockSpec((128, _PACK), lambda i: (0, 0)),
        pl.BlockSpec(memory_space=pltpu.SMEM),
    ],
    out_specs=pl.BlockSpec((_BLK, _PACK), lambda i: (i, 0)),
    out_shape=jax.ShapeDtypeStruct((_ROWS, _PACK), jnp.float32),
)


_CHUNK = _IDX_ROWS * 128
_NBUF = 2
_J = HIST // 16
_TAIL = HIST - _J * 16


def _pool_body(xr_hbm, s_hbm, out_hbm, idx_v, g0, g1, out_v, sem0, sem1):
    g = [g0, g1]
    sem = [sem0, sem1]
    wid = lax.axis_index("s") * _NC + lax.axis_index("c")
    base = wid * _BPW

    pltpu.sync_copy(
        xr_hbm.at[pl.ds(base * _IDX_ROWS, _BPW * _IDX_ROWS), :], idx_v
    )

    def fire(blk, k):
        row0 = blk * _IDX_ROWS
        for i in range(_IDX_ROWS):
            pltpu.async_copy(
                s_hbm.at[idx_v.at[row0 + i]],
                g[k].at[pl.ds(i * 128, 128)],
                sem[k],
            )

    def wait_g(k):
        pltpu.make_async_copy(
            s_hbm.at[pl.ds(0, _CHUNK)], g[k].at[pl.ds(0, _CHUNK)], sem[k]
        ).wait()

    for k in range(_NBUF):
        fire(k, k)

    def body(it, carry):
        blk0 = it * _NBUF
        for k in range(_NBUF):
            blk = blk0 + k
            wait_g(k)
            acc = jnp.zeros((16,), jnp.float32)
            for i in range(_IDX_ROWS):
                for c in range(_PACK):
                    acc = acc + g[k][pl.ds(i * 128 + c * 16, 16)]
            m = acc * (1.0 / HIST)
            y = 1.0 / (1.0 + jnp.exp(-m))
            y = (y * 100.0 + 0.5).astype(jnp.int32).astype(jnp.float32) / 100.0
            out_v[pl.ds(blk * _RB, _RB)] = y
            nxt = blk + _NBUF

            @pl.when(nxt < _BPW)
            def _():
                fire(nxt, k)

        return carry

    lax.fori_loop(0, _BPW // _NBUF, body, 0)
    pltpu.sync_copy(out_v, out_hbm.at[pl.ds(base * _RB, _BPW * _RB)])


@functools.cache
def _pool():
    return pl.kernel(
        _pool_body,
        out_type=jax.ShapeDtypeStruct((BATCH,), jnp.float32),
        mesh=plsc.VectorSubcoreMesh(
            core_axis_name="c", subcore_axis_name="s",
            num_cores=_NC, num_subcores=_NS,
        ),
        scratch_types=[
            pltpu.VMEM((_BPW * _IDX_ROWS, 128), jnp.int32),
            pltpu.VMEM((_CHUNK,), jnp.float32),
            pltpu.VMEM((_CHUNK,), jnp.float32),
            pltpu.VMEM((_BPW * _RB,), jnp.float32),
            pltpu.SemaphoreType.DMA,
            pltpu.SemaphoreType.DMA,
        ],
    )


def kernel(x, table, W, b):
    xr = (
        x.astype(jnp.int32)
        .reshape(_NB, _RB, HIST)
        .transpose(0, 2, 1)
        .reshape(_NB * _IDX_ROWS, 128)
    )
    sel = jnp.repeat(jnp.eye(_PACK, dtype=jnp.float32), EMBED_DIM, axis=0)
    sel = sel * jnp.tile(W.reshape(EMBED_DIM), _PACK)[:, None]
    scores = _score(table.reshape(_ROWS, 128), sel, b.reshape(1, 1))
    y = _pool()(xr, scores.reshape(VOCAB))
    return y.reshape(BATCH, 1)
